# baseline (speedup 1.0000x reference)
import jax
import jax.numpy as jnp
from jax import lax
from jax.experimental import pallas as pl
from jax.experimental.pallas import tpu as pltpu
from jax.experimental.pallas import tpu_sc as plsc

B, L, UNITS = 4096, 200, 64
N = B * L
NC, NS = 2, 16
NW = NC * NS
BLK = B // NW
PER_W = BLK * L
TAB = 256 * UNITS
SLAB = 8 * 8 * 128


def _body(x_hbm, w_hbm, out_hbm, wt, dt, xb, idxb, z_bufs, so):
    wid = lax.axis_index("s") * NC + lax.axis_index("c")
    pltpu.sync_copy(w_hbm, wt)

    @plsc.parallel_loop(0, (TAB - UNITS) // 16, unroll=4)
    def diff_body(k):
        dt[pl.ds(k * 16, 16)] = wt[pl.ds(k * 16 + UNITS, 16)] - wt[pl.ds(k * 16, 16)]

    for j in range(4):
        dt[pl.ds(TAB - UNITS + j * 16, 16)] = jnp.zeros((16,), jnp.float32)

    pltpu.sync_copy(x_hbm.at[pl.ds(wid * PER_W, PER_W)], xb)

    @plsc.parallel_loop(0, PER_W // 16, unroll=4)
    def prepass(g):
        v = xb[pl.ds(g * 16, 16)]
        iv = v.astype(jnp.int32)
        idxb[pl.ds(g * 16, 16)] = iv * UNITS
        xb[pl.ds(g * 16, 16)] = v - iv.astype(jnp.float32)

    iota_l = lax.iota(jnp.int32, 16) * L

    def process_l(l, zb):
        offs, als = [], []
        for g in range(8):
            ixv = iota_l + (g * 16 * L + l)
            offs.append(plsc.load_gather(idxb, [ixv]))
            als.append(plsc.load_gather(xb, [ixv]))

        @plsc.parallel_loop(0, UNITS, unroll=4)
        def u_loop(u):
            for g in range(8):
                ix = offs[g] + u
                lo = plsc.load_gather(wt, [ix])
                d = plsc.load_gather(dt, [ix])
                zb[pl.ds(u * 128 + g * 16, 16)] = lo + als[g] * d

    def l_pair(k, _):
        for p in range(2):
            l = 2 * k + p
            zb = z_bufs[p]

            @pl.when(k > 0)
            def _wait_prev_store():
                for ub in range(8):
                    pltpu.make_async_copy(
                        zb.at[pl.ds(ub * 1024, 1024)],
                        out_hbm.at[pl.ds(ub * 1024, 1024)],
                        so[p],
                    ).wait()

            process_l(l, zb)
            for ub in range(8):
                pltpu.async_copy(
                    zb.at[pl.ds(ub * 1024, 1024)],
                    out_hbm.at[pl.ds((l * 8 + ub) * (NW * 1024) + wid * 1024, 1024)],
                    so[p],
                )
        return 0

    lax.fori_loop(0, L // 2, l_pair, 0)

    for p in range(2):
        for ub in range(8):
            pltpu.make_async_copy(
                z_bufs[p].at[pl.ds(ub * 1024, 1024)],
                out_hbm.at[pl.ds(ub * 1024, 1024)],
                so[p],
            ).wait()


@jax.jit
def _run(x_flat, w_flat):
    mesh = plsc.VectorSubcoreMesh(core_axis_name="c", subcore_axis_name="s")
    return pl.kernel(
        _body,
        out_type=jax.ShapeDtypeStruct((N * UNITS,), jnp.float32),
        mesh=mesh,
        compiler_params=pltpu.CompilerParams(needs_layout_passes=False),
        scratch_types=[
            pltpu.VMEM((TAB,), jnp.float32),
            pltpu.VMEM((TAB,), jnp.float32),
            pltpu.VMEM((PER_W,), jnp.float32),
            pltpu.VMEM((PER_W,), jnp.int32),
            [pltpu.VMEM((SLAB,), jnp.float32)] * 2,
            [pltpu.SemaphoreType.DMA] * 2,
        ],
    )(x_flat, w_flat)


def kernel(inputs, W):
    x_flat = inputs.reshape(N)
    w_flat = W.reshape(TAB)
    out = _run(x_flat, w_flat)
    z = out.reshape(L, 8, NW, 8, 128)
    return z.transpose(2, 4, 0, 1, 3).reshape(B, L, 1, UNITS)


# speedup vs baseline: 2.7009x; 2.7009x over previous
import jax
import jax.numpy as jnp
from jax import lax
from jax.experimental import layout as jax_layout
from jax.experimental import pallas as pl
from jax.experimental.pallas import tpu as pltpu
from jax.experimental.pallas import tpu_sc as plsc

B, L, UNITS = 4096, 200, 64
N = B * L
NC, NS = 2, 16
NW = NC * NS
PER_W = N // NW
C = 512
N_CHUNKS = PER_W // C
TAB = 256 * UNITS


def _body(x_hbm, w_hbm, out_hbm, wt, dt, x_bufs, o_bufs, sx, so):
    wid = lax.axis_index("s") * NC + lax.axis_index("c")
    start = wid * PER_W
    pltpu.sync_copy(w_hbm, wt)

    iota = lax.iota(jnp.int32, 16)

    @plsc.parallel_loop(0, (TAB - UNITS) // 16, unroll=4)
    def diff_body(k):
        dt[pl.ds(k * 16, 16)] = wt[pl.ds(k * 16 + UNITS, 16)] - wt[pl.ds(k * 16, 16)]

    for j in range(4):
        dt[pl.ds(TAB - UNITS + j * 16, 16)] = jnp.zeros((16,), jnp.float32)

    def compute(xb, ob):
        @plsc.parallel_loop(0, C // 16, unroll=4)
        def lerp_body(g):
            v = xb[pl.ds(g * 16, 16)]
            iv = v.astype(jnp.int32)
            alv = v - iv.astype(jnp.float32)
            offv = iv * UNITS
            for lane in range(16):
                idx = jnp.full((16,), offv[lane], jnp.int32) + iota
                av = jnp.full((16,), alv[lane], jnp.float32)
                out_base = (g * 16 + lane) * UNITS
                idxs = [idx + 16 * j for j in range(4)]
                los = [plsc.load_gather(wt, [ix]) for ix in idxs]
                dvs = [plsc.load_gather(dt, [ix]) for ix in idxs]
                for j in range(4):
                    ob[pl.ds(out_base + 16 * j, 16)] = los[j] + av * dvs[j]

    for b in range(2):
        pltpu.async_copy(x_hbm.at[pl.ds(start + b * C, C)], x_bufs[b], sx[b])

    def chunk_pair(k, _):
        for b in range(2):
            ci = 2 * k + b
            base = start + ci * C
            xb, ob = x_bufs[b], o_bufs[b]

            @pl.when(k > 0)
            def _wait_prev_store():
                pltpu.make_async_copy(
                    ob, out_hbm.at[pl.ds(base * UNITS, C * UNITS)], so[b]
                ).wait()

            pltpu.make_async_copy(x_hbm.at[pl.ds(base, C)], xb, sx[b]).wait()
            compute(xb, ob)
            pltpu.async_copy(
                ob, out_hbm.at[pl.ds(base * UNITS, C * UNITS)], so[b]
            )

            @pl.when(ci + 2 < N_CHUNKS)
            def _prefetch_next():
                pltpu.async_copy(
                    x_hbm.at[pl.ds(base + 2 * C, C)], xb, sx[b]
                )

        return 0

    lax.fori_loop(0, N_CHUNKS // 2, chunk_pair, 0)

    for b in range(2):
        pltpu.make_async_copy(
            o_bufs[b], out_hbm.at[pl.ds(start * UNITS, C * UNITS)], so[b]
        ).wait()


@jax.jit
def _run(x_flat, w_flat):
    mesh = plsc.VectorSubcoreMesh(core_axis_name="c", subcore_axis_name="s")
    return pl.kernel(
        _body,
        out_type=jax.ShapeDtypeStruct((N * UNITS,), jnp.float32),
        mesh=mesh,
        compiler_params=pltpu.CompilerParams(needs_layout_passes=False),
        scratch_types=[
            pltpu.VMEM((TAB,), jnp.float32),
            pltpu.VMEM((TAB,), jnp.float32),
            [pltpu.VMEM((C,), jnp.float32)] * 2,
            [pltpu.VMEM((C * UNITS,), jnp.float32)] * 2,
            [pltpu.SemaphoreType.DMA] * 2,
            [pltpu.SemaphoreType.DMA] * 2,
        ],
    )(x_flat, w_flat)


def kernel(inputs, W):
    x_flat = inputs.reshape(N)
    w_flat = W.reshape(TAB)
    out = _run(x_flat, w_flat)
    out4 = out.reshape(B, L, 1, UNITS)
    return jax_layout.with_layout_constraint(
        out4, jax_layout.Layout(major_to_minor=(2, 0, 1, 3))
    )


# speedup vs baseline: 2.7202x; 1.0072x over previous
import jax
import jax.numpy as jnp
from jax import lax
from jax.experimental import layout as jax_layout
from jax.experimental import pallas as pl
from jax.experimental.pallas import tpu as pltpu
from jax.experimental.pallas import tpu_sc as plsc

B, L, UNITS = 4096, 200, 64
N = B * L
NC, NS = 2, 16
NW = NC * NS
PER_W = N // NW
C = 512
N_CHUNKS = PER_W // C
TAB = 256 * UNITS


def _body(x_hbm, w_hbm, out_hbm, wt, dt, x_bufs, o_bufs, sx, so):
    wid = lax.axis_index("s") * NC + lax.axis_index("c")
    start = wid * PER_W
    pltpu.sync_copy(w_hbm, wt)

    iota = lax.iota(jnp.int32, 16)

    @plsc.parallel_loop(0, (TAB - UNITS) // 16, unroll=4)
    def diff_body(k):
        dt[pl.ds(k * 16, 16)] = wt[pl.ds(k * 16 + UNITS, 16)] - wt[pl.ds(k * 16, 16)]

    for j in range(4):
        dt[pl.ds(TAB - UNITS + j * 16, 16)] = jnp.zeros((16,), jnp.float32)

    def compute(xb, ob):
        @plsc.parallel_loop(0, C // 16, unroll=4)
        def lerp_body(g):
            v = xb[pl.ds(g * 16, 16)]
            iv = v.astype(jnp.int32)
            alv = v - iv.astype(jnp.float32)
            offv = iv * UNITS
            for lane in range(16):
                idx = jnp.full((16,), offv[lane], jnp.int32) + iota
                av = jnp.full((16,), alv[lane], jnp.float32)
                out_base = (g * 16 + lane) * UNITS
                idxs = [idx + 16 * j for j in range(4)]
                los = [plsc.load_gather(wt, [ix]) for ix in idxs]
                dvs = [plsc.load_gather(dt, [ix]) for ix in idxs]
                for j in range(4):
                    ob[pl.ds(out_base + 16 * j, 16)] = los[j] + av * dvs[j]

    for b in range(2):
        pltpu.async_copy(x_hbm.at[pl.ds(start + b * C, C)], x_bufs[b], sx[b])

    def chunk_pair(k, _):
        for b in range(2):
            ci = 2 * k + b
            base = start + ci * C
            xb, ob = x_bufs[b], o_bufs[b]

            @pl.when(k > 0)
            def _wait_prev_store():
                pltpu.make_async_copy(
                    ob, out_hbm.at[pl.ds(base * UNITS, C * UNITS)], so[b]
                ).wait()

            pltpu.make_async_copy(x_hbm.at[pl.ds(base, C)], xb, sx[b]).wait()
            compute(xb, ob)
            pltpu.async_copy(
                ob, out_hbm.at[pl.ds(base * UNITS, C * UNITS)], so[b]
            )

            @pl.when(ci + 2 < N_CHUNKS)
            def _prefetch_next():
                pltpu.async_copy(
                    x_hbm.at[pl.ds(base + 2 * C, C)], xb, sx[b]
                )

        return 0

    lax.fori_loop(0, N_CHUNKS // 2, chunk_pair, 0)

    for b in range(2):
        pltpu.make_async_copy(
            o_bufs[b], out_hbm.at[pl.ds(start * UNITS, C * UNITS)], so[b]
        ).wait()


@jax.jit
def _run(x_flat, w_flat):
    mesh = plsc.VectorSubcoreMesh(core_axis_name="c", subcore_axis_name="s")
    return pl.kernel(
        _body,
        out_type=jax.ShapeDtypeStruct((N * UNITS,), jnp.float32),
        mesh=mesh,
        compiler_params=pltpu.CompilerParams(needs_layout_passes=False),
        scratch_types=[
            pltpu.VMEM((TAB,), jnp.float32),
            pltpu.VMEM((TAB,), jnp.float32),
            [pltpu.VMEM((C,), jnp.float32)] * 2,
            [pltpu.VMEM((C * UNITS,), jnp.float32)] * 2,
            [pltpu.SemaphoreType.DMA] * 2,
            [pltpu.SemaphoreType.DMA] * 2,
        ],
    )(x_flat, w_flat)


def kernel(inputs, W):
    x_flat = inputs.reshape(N)
    w_flat = W.reshape(TAB)
    out = _run(x_flat, w_flat)
    out4 = out.reshape(B, L, 1, UNITS)
    return jax_layout.with_layout_constraint(
        out4, jax_layout.Layout(major_to_minor=(0, 1, 2, 3), tiling=((1024,),))
    )
